# baseline (speedup 1.0000x reference)
import jax
import jax.numpy as jnp
from jax import lax
from jax.experimental import pallas as pl
from jax.experimental.pallas import tpu as pltpu

_CHUNK = 4096
_NBUF = 4


def _body(x_hbm, wt_ref, b_ref, o_hbm, x_buf, o_buf, in_sems, out_sems):
    n = x_hbm.shape[0]
    num_chunks = n // _CHUNK
    wt = wt_ref[...]
    b_col = jnp.transpose(b_ref[...])

    def start_in(c, slot):
        pltpu.make_async_copy(
            x_hbm.at[pl.ds(c * _CHUNK, _CHUNK), :],
            x_buf.at[slot],
            in_sems.at[slot],
        ).start()

    for s in range(_NBUF):
        start_in(s, s)

    def step(c, _):
        slot = jax.lax.rem(c, _NBUF)
        pltpu.make_async_copy(
            x_hbm.at[pl.ds(c * _CHUNK, _CHUNK), :],
            x_buf.at[slot],
            in_sems.at[slot],
        ).wait()

        @pl.when(c >= _NBUF)
        def _():
            pltpu.make_async_copy(
                o_buf.at[slot],
                o_hbm.at[:, pl.ds((c - _NBUF) * _CHUNK, _CHUNK)],
                out_sems.at[slot],
            ).wait()

        o_buf[slot] = (
            lax.dot_general(
                wt,
                x_buf[slot],
                dimension_numbers=(((1,), (1,)), ((), ())),
                preferred_element_type=jnp.float32,
            )
            + b_col
        )
        pltpu.make_async_copy(
            o_buf.at[slot],
            o_hbm.at[:, pl.ds(c * _CHUNK, _CHUNK)],
            out_sems.at[slot],
        ).start()

        @pl.when(c + _NBUF < num_chunks)
        def _():
            start_in(c + _NBUF, slot)

        return _

    jax.lax.fori_loop(0, num_chunks, step, None)

    for s in range(_NBUF):
        c = num_chunks - _NBUF + s
        slot = jax.lax.rem(jnp.int32(c), _NBUF)
        pltpu.make_async_copy(
            o_buf.at[slot],
            o_hbm.at[:, pl.ds(c * _CHUNK, _CHUNK)],
            out_sems.at[slot],
        ).wait()


def kernel(input, W, b):
    n, in_f = input.shape
    out_f = W.shape[1]
    wt = W.T
    b_row = b.reshape(1, out_f)
    out_t = pl.pallas_call(
        _body,
        in_specs=[
            pl.BlockSpec(memory_space=pl.ANY),
            pl.BlockSpec(memory_space=pltpu.VMEM),
            pl.BlockSpec(memory_space=pltpu.VMEM),
        ],
        out_specs=pl.BlockSpec(memory_space=pl.ANY),
        out_shape=jax.ShapeDtypeStruct((out_f, n), jnp.float32),
        scratch_shapes=[
            pltpu.VMEM((_NBUF, _CHUNK, in_f), jnp.float32),
            pltpu.VMEM((_NBUF, out_f, _CHUNK), jnp.float32),
            pltpu.SemaphoreType.DMA((_NBUF,)),
            pltpu.SemaphoreType.DMA((_NBUF,)),
        ],
    )(input, wt, b_row)
    return out_t.T


# speedup vs baseline: 1.0035x; 1.0035x over previous
import jax
import jax.numpy as jnp
from jax import lax
from jax.experimental import pallas as pl
from jax.experimental.pallas import tpu as pltpu

_CHUNK = 1024
_NBUF = 16


def _body(x_hbm, wt_ref, b_ref, o_hbm, x_buf, o_buf, in_sems, out_sems):
    n = x_hbm.shape[0]
    num_chunks = n // _CHUNK
    wt = wt_ref[...]
    b_col = jnp.transpose(b_ref[...])

    def start_in(c, slot):
        pltpu.make_async_copy(
            x_hbm.at[pl.ds(c * _CHUNK, _CHUNK), :],
            x_buf.at[slot],
            in_sems.at[slot],
        ).start()

    for s in range(_NBUF):
        start_in(s, s)

    def step(c, _):
        slot = jax.lax.rem(c, _NBUF)
        pltpu.make_async_copy(
            x_hbm.at[pl.ds(c * _CHUNK, _CHUNK), :],
            x_buf.at[slot],
            in_sems.at[slot],
        ).wait()

        @pl.when(c >= _NBUF)
        def _():
            pltpu.make_async_copy(
                o_buf.at[slot],
                o_hbm.at[:, pl.ds((c - _NBUF) * _CHUNK, _CHUNK)],
                out_sems.at[slot],
            ).wait()

        o_buf[slot] = (
            lax.dot_general(
                wt,
                x_buf[slot],
                dimension_numbers=(((1,), (1,)), ((), ())),
                preferred_element_type=jnp.float32,
            )
            + b_col
        )
        pltpu.make_async_copy(
            o_buf.at[slot],
            o_hbm.at[:, pl.ds(c * _CHUNK, _CHUNK)],
            out_sems.at[slot],
        ).start()

        @pl.when(c + _NBUF < num_chunks)
        def _():
            start_in(c + _NBUF, slot)

        return _

    jax.lax.fori_loop(0, num_chunks, step, None)

    for s in range(_NBUF):
        c = num_chunks - _NBUF + s
        slot = jax.lax.rem(jnp.int32(c), _NBUF)
        pltpu.make_async_copy(
            o_buf.at[slot],
            o_hbm.at[:, pl.ds(c * _CHUNK, _CHUNK)],
            out_sems.at[slot],
        ).wait()


def kernel(input, W, b):
    n, in_f = input.shape
    out_f = W.shape[1]
    wt = W.T
    b_row = b.reshape(1, out_f)
    out_t = pl.pallas_call(
        _body,
        in_specs=[
            pl.BlockSpec(memory_space=pl.ANY),
            pl.BlockSpec(memory_space=pltpu.VMEM),
            pl.BlockSpec(memory_space=pltpu.VMEM),
        ],
        out_specs=pl.BlockSpec(memory_space=pl.ANY),
        out_shape=jax.ShapeDtypeStruct((out_f, n), jnp.float32),
        scratch_shapes=[
            pltpu.VMEM((_NBUF, _CHUNK, in_f), jnp.float32),
            pltpu.VMEM((_NBUF, out_f, _CHUNK), jnp.float32),
            pltpu.SemaphoreType.DMA((_NBUF,)),
            pltpu.SemaphoreType.DMA((_NBUF,)),
        ],
    )(input, wt, b_row)
    return out_t.T


# speedup vs baseline: 1.0091x; 1.0056x over previous
import jax
import jax.numpy as jnp
from jax import lax
from jax.experimental import pallas as pl
from jax.experimental.pallas import tpu as pltpu

_CHUNK = 2048
_NBUF = 8


def _body(x_hbm, wt_ref, b_ref, o_hbm, x_buf, o_buf, in_sems, out_sems):
    n = x_hbm.shape[0]
    num_chunks = n // _CHUNK
    wt = wt_ref[...]
    b_col = jnp.transpose(b_ref[...])

    def start_in(c, slot):
        pltpu.make_async_copy(
            x_hbm.at[pl.ds(c * _CHUNK, _CHUNK), :],
            x_buf.at[slot],
            in_sems.at[slot],
        ).start()

    for s in range(_NBUF):
        start_in(s, s)

    def step(c, _):
        slot = jax.lax.rem(c, _NBUF)
        pltpu.make_async_copy(
            x_hbm.at[pl.ds(c * _CHUNK, _CHUNK), :],
            x_buf.at[slot],
            in_sems.at[slot],
        ).wait()

        @pl.when(c >= _NBUF)
        def _():
            pltpu.make_async_copy(
                o_buf.at[slot],
                o_hbm.at[:, pl.ds((c - _NBUF) * _CHUNK, _CHUNK)],
                out_sems.at[slot],
            ).wait()

        o_buf[slot] = (
            lax.dot_general(
                wt,
                x_buf[slot],
                dimension_numbers=(((1,), (1,)), ((), ())),
                preferred_element_type=jnp.float32,
            )
            + b_col
        )
        pltpu.make_async_copy(
            o_buf.at[slot],
            o_hbm.at[:, pl.ds(c * _CHUNK, _CHUNK)],
            out_sems.at[slot],
        ).start()

        @pl.when(c + _NBUF < num_chunks)
        def _():
            start_in(c + _NBUF, slot)

        return _

    jax.lax.fori_loop(0, num_chunks, step, None)

    for s in range(_NBUF):
        c = num_chunks - _NBUF + s
        slot = jax.lax.rem(jnp.int32(c), _NBUF)
        pltpu.make_async_copy(
            o_buf.at[slot],
            o_hbm.at[:, pl.ds(c * _CHUNK, _CHUNK)],
            out_sems.at[slot],
        ).wait()


def kernel(input, W, b):
    n, in_f = input.shape
    out_f = W.shape[1]
    wt = W.T
    b_row = b.reshape(1, out_f)
    out_t = pl.pallas_call(
        _body,
        in_specs=[
            pl.BlockSpec(memory_space=pl.ANY),
            pl.BlockSpec(memory_space=pltpu.VMEM),
            pl.BlockSpec(memory_space=pltpu.VMEM),
        ],
        out_specs=pl.BlockSpec(memory_space=pl.ANY),
        out_shape=jax.ShapeDtypeStruct((out_f, n), jnp.float32),
        scratch_shapes=[
            pltpu.VMEM((_NBUF, _CHUNK, in_f), jnp.float32),
            pltpu.VMEM((_NBUF, out_f, _CHUNK), jnp.float32),
            pltpu.SemaphoreType.DMA((_NBUF,)),
            pltpu.SemaphoreType.DMA((_NBUF,)),
        ],
    )(input, wt, b_row)
    return out_t.T
